# baseline (speedup 1.0000x reference)
import functools

import jax
import jax.numpy as jnp
from jax import lax
from jax.experimental import pallas as pl
from jax.experimental.pallas import tpu as pltpu
from jax.experimental.pallas import tpu_sc as plsc

B = 16384
NW = 32
BPW = B // NW


def _sc_gather(user_idx, item_idx, user_emb, item_emb):
    mesh = plsc.VectorSubcoreMesh(core_axis_name="c", subcore_axis_name="s")

    @functools.partial(
        pl.kernel,
        mesh=mesh,
        compiler_params=pltpu.CompilerParams(use_tc_tiling_on_sc=False),
        out_type=(
            jax.ShapeDtypeStruct((B, 3), jnp.float32),
            jax.ShapeDtypeStruct((B, 3), jnp.float32),
        ),
        scratch_types=[
            pltpu.VMEM((BPW,), jnp.int32),
            pltpu.VMEM((BPW, 3), jnp.float32),
            pltpu.VMEM((BPW,), jnp.int32),
            pltpu.VMEM((BPW, 3), jnp.float32),
            pltpu.SemaphoreType.DMA,
            pltpu.SemaphoreType.DMA,
        ],
    )
    def gather_kernel(uidx_hbm, iidx_hbm, uemb_hbm, iemb_hbm,
                      uout_hbm, iout_hbm,
                      uidx_v, urows_v, iidx_v, irows_v, sem_u, sem_i):
        wid = lax.axis_index("s") * 2 + lax.axis_index("c")
        base = wid * BPW
        pltpu.sync_copy(uidx_hbm.at[pl.ds(base, BPW)], uidx_v)
        pltpu.sync_copy(iidx_hbm.at[pl.ds(base, BPW)], iidx_v)
        cu = pltpu.async_copy(uemb_hbm.at[uidx_v], urows_v, sem_u)
        ci = pltpu.async_copy(iemb_hbm.at[iidx_v], irows_v, sem_i)
        cu.wait()
        ci.wait()
        pltpu.sync_copy(urows_v, uout_hbm.at[pl.ds(base, BPW)])
        pltpu.sync_copy(irows_v, iout_hbm.at[pl.ds(base, BPW)])

    return gather_kernel(user_idx, item_idx, user_emb, item_emb)


def _lrelu(v):
    return jnp.where(v >= 0, v, 0.1 * v)


def _mlp_body(u_ref, i_ref, a_ref, w1u, w1i, w1a, b1, w2, b2, w3, b3,
              w4, b4, w5, b5, w6, b6, w7, b7, out_ref):
    f32 = jnp.float32
    dot = functools.partial(jnp.dot, preferred_element_type=f32,
                            precision=lax.Precision.HIGHEST)
    h = (dot(u_ref[...], w1u[...]) + dot(i_ref[...], w1i[...])
         + dot(a_ref[...], w1a[...]) + b1[...])
    h = _lrelu(h)
    h = _lrelu(dot(h, w2[...]) + b2[...])
    h = _lrelu(dot(h, w3[...]) + b3[...])
    h = dot(h, w4[...]) + b4[...]
    h = _lrelu(dot(h, w5[...]) + b5[...])
    h = _lrelu(dot(h, w6[...]) + b6[...])
    h = dot(h, w7[...]) + b7[...]
    out_ref[...] = 5.0 / (1.0 + jnp.exp(-h))


def _mlp(u, i, a, *ws, tb=2048):
    def _full(arr):
        return pl.BlockSpec(arr.shape, lambda j: (0,) * arr.ndim)

    in_specs = [
        pl.BlockSpec((tb, 3), lambda j: (j, 0)),
        pl.BlockSpec((tb, 3), lambda j: (j, 0)),
        pl.BlockSpec((tb, 5), lambda j: (j, 0)),
    ] + [_full(w) for w in ws]
    return pl.pallas_call(
        _mlp_body,
        grid=(B // tb,),
        in_specs=in_specs,
        out_specs=pl.BlockSpec((tb, 1), lambda j: (j, 0)),
        out_shape=jax.ShapeDtypeStruct((B, 1), jnp.float32),
    )(u, i, a, *ws)


def kernel(x, a, user_emb, item_emb, W1, b1, W2, b2, W3, b3, W4, b4,
           W5, b5, W6, b6, W7, b7):
    user_idx = x[:, 0]
    item_idx = x[:, 1]
    u, i = _sc_gather(user_idx, item_idx, user_emb, item_emb)
    out = _mlp(
        u, i, a,
        W1[0:3], W1[3:6], W1[6:11], b1.reshape(1, 10),
        W2, b2.reshape(1, 10), W3, b3.reshape(1, 10),
        W4, b4.reshape(1, 10), W5, b5.reshape(1, 10),
        W6, b6.reshape(1, 10), W7, b7.reshape(1, 1),
    )
    return out[:, 0]


# speedup vs baseline: 59.6674x; 59.6674x over previous
import functools

import jax
import jax.numpy as jnp
from jax import lax
from jax.experimental import pallas as pl
from jax.experimental.pallas import tpu as pltpu
from jax.experimental.pallas import tpu_sc as plsc

B = 16384
NW = 32
BPW = B // NW
NU = 1000000


def _sc_gather_t(uflat_idx, iflat_idx, uembf, iembf):
    mesh = plsc.VectorSubcoreMesh(core_axis_name="c", subcore_axis_name="s")

    @functools.partial(
        pl.kernel,
        mesh=mesh,
        compiler_params=pltpu.CompilerParams(use_tc_tiling_on_sc=False),
        out_type=(
            jax.ShapeDtypeStruct((3, B), jnp.float32),
            jax.ShapeDtypeStruct((3, B), jnp.float32),
        ),
        scratch_types=[
            pltpu.VMEM((3, BPW), jnp.int32),
            pltpu.VMEM((3, BPW), jnp.float32),
            pltpu.VMEM((3, BPW), jnp.int32),
            pltpu.VMEM((3, BPW), jnp.float32),
            pltpu.SemaphoreType.DMA,
        ],
    )
    def gather_kernel(uidx_hbm, iidx_hbm, uembf_hbm, iembf_hbm,
                      uout_hbm, iout_hbm,
                      uidx_v, urow_v, iidx_v, irow_v, sem):
        wid = lax.axis_index("s") * 2 + lax.axis_index("c")
        base = wid * BPW
        for c in range(3):
            pltpu.sync_copy(uidx_hbm.at[c, pl.ds(base, BPW)], uidx_v.at[c])
            pltpu.sync_copy(iidx_hbm.at[c, pl.ds(base, BPW)], iidx_v.at[c])
        copies = []
        for c in range(3):
            copies.append(
                pltpu.async_copy(uembf_hbm.at[uidx_v.at[c]], urow_v.at[c], sem))
            copies.append(
                pltpu.async_copy(iembf_hbm.at[iidx_v.at[c]], irow_v.at[c], sem))
        for cp in copies:
            cp.wait()
        for c in range(3):
            pltpu.sync_copy(urow_v.at[c], uout_hbm.at[c, pl.ds(base, BPW)])
            pltpu.sync_copy(irow_v.at[c], iout_hbm.at[c, pl.ds(base, BPW)])

    return gather_kernel(uflat_idx, iflat_idx, uembf, iembf)


def _lrelu(v):
    return jnp.where(v >= 0, v, 0.1 * v)


def _mlp_t_body(u_ref, i_ref, a_ref, w1u, w1i, w1a, b1, w2, b2, w3, b3,
                w4, b4, w5, b5, w6, b6, w7, b7, out_ref):
    dot = functools.partial(jnp.dot, preferred_element_type=jnp.float32,
                            precision=lax.Precision.HIGHEST)
    h = (dot(w1u[...], u_ref[...]) + dot(w1i[...], i_ref[...])
         + dot(w1a[...], a_ref[...]) + b1[...])
    h = _lrelu(h)
    h = _lrelu(dot(w2[...], h) + b2[...])
    h = _lrelu(dot(w3[...], h) + b3[...])
    h = dot(w4[...], h) + b4[...]
    h = _lrelu(dot(w5[...], h) + b5[...])
    h = _lrelu(dot(w6[...], h) + b6[...])
    h = dot(w7[...], h) + b7[...]
    out_ref[...] = 5.0 / (1.0 + jnp.exp(-h))


def _mlp_t(u, i, a, *ws, tb=4096):
    def _full(arr):
        return pl.BlockSpec(arr.shape, lambda j: (0,) * arr.ndim)

    in_specs = [
        pl.BlockSpec((3, tb), lambda j: (0, j)),
        pl.BlockSpec((3, tb), lambda j: (0, j)),
        pl.BlockSpec((5, tb), lambda j: (0, j)),
    ] + [_full(w) for w in ws]
    return pl.pallas_call(
        _mlp_t_body,
        grid=(B // tb,),
        in_specs=in_specs,
        out_specs=pl.BlockSpec((1, tb), lambda j: (0, j)),
        out_shape=jax.ShapeDtypeStruct((1, B), jnp.float32),
    )(u, i, a, *ws)


def kernel(x, a, user_emb, item_emb, W1, b1, W2, b2, W3, b3, W4, b4,
           W5, b5, W6, b6, W7, b7):
    user_idx = x[:, 0]
    item_idx = x[:, 1]
    off = jnp.arange(3, dtype=jnp.int32)[:, None] * NU
    uflat_idx = user_idx[None, :] + off
    iflat_idx = item_idx[None, :] + off
    uembf = user_emb.T.reshape(-1)
    iembf = item_emb.T.reshape(-1)
    u, i = _sc_gather_t(uflat_idx, iflat_idx, uembf, iembf)
    out = _mlp_t(
        u, i, a.T,
        W1[0:3].T, W1[3:6].T, W1[6:11].T, b1.reshape(10, 1),
        W2.T, b2.reshape(10, 1), W3.T, b3.reshape(10, 1),
        W4.T, b4.reshape(10, 1), W5.T, b5.reshape(10, 1),
        W6.T, b6.reshape(10, 1), W7.T, b7.reshape(1, 1),
    )
    return out[0]
